# Initial kernel scaffold; baseline (speedup 1.0000x reference)
import jax
import jax.numpy as jnp
from jax.experimental import pallas as pl


def kernel(x, edge_index, W, b):
    raise NotImplementedError("write your pallas kernel here")


# speedup vs baseline: 16.1669x; 16.1669x over previous
import functools

import jax
import jax.numpy as jnp
from jax import lax
from jax.experimental import pallas as pl
from jax.experimental.pallas import tpu as pltpu
from jax.experimental.pallas import tpu_sc as plsc

N_CORES = 2
N_SUBCORES = 16
N_WORKERS = N_CORES * N_SUBCORES
CHUNK = 128
LANES = 16


def _sc_mesh():
    return plsc.VectorSubcoreMesh(core_axis_name="c", subcore_axis_name="s")


def _make_deg_kernel(e_pad, chunks_per_worker, n_acc, rows_per_sub):
    zero_blocks = rows_per_sub // CHUNK

    @functools.partial(
        pl.kernel,
        out_type=jax.ShapeDtypeStruct((N_CORES, n_acc), jnp.float32),
        mesh=_sc_mesh(),
        scratch_types=[
            pltpu.VMEM((CHUNK,), jnp.int32),
            pltpu.VMEM((CHUNK,), jnp.float32),
            pltpu.VMEM_SHARED((n_acc,), jnp.float32),
        ],
    )
    def deg_kernel(col_hbm, out_hbm, cidx_v, val_v, acc_sh):
        c = lax.axis_index("c")
        s = lax.axis_index("s")

        def fill(i, _):
            val_v[pl.ds(i * LANES, LANES)] = jnp.zeros((LANES,), jnp.float32)
            return 0

        lax.fori_loop(0, CHUNK // LANES, fill, 0)
        for t in range(zero_blocks):
            pltpu.sync_copy(
                val_v, acc_sh.at[pl.ds(s * rows_per_sub + t * CHUNK, CHUNK)]
            )
        plsc.subcore_barrier()

        def ones(i, _):
            val_v[pl.ds(i * LANES, LANES)] = jnp.ones((LANES,), jnp.float32)
            return 0

        lax.fori_loop(0, CHUNK // LANES, ones, 0)

        w = c * N_SUBCORES + s

        def body(j, _):
            base = w * (chunks_per_worker * CHUNK) + j * CHUNK
            pltpu.sync_copy(col_hbm.at[pl.ds(base, CHUNK)], cidx_v)
            pltpu.sync_copy(val_v, acc_sh.at[cidx_v], add=True)
            return 0

        lax.fori_loop(0, chunks_per_worker, body, 0)
        plsc.subcore_barrier()
        pltpu.sync_copy(
            acc_sh.at[pl.ds(s * rows_per_sub, rows_per_sub)],
            out_hbm.at[c, pl.ds(s * rows_per_sub, rows_per_sub)],
        )

    return deg_kernel


def _make_scatter_kernel(e_pad, chunks_per_worker, n_acc, rows_per_sub, d):
    zero_blocks = rows_per_sub // CHUNK

    @functools.partial(
        pl.kernel,
        out_type=jax.ShapeDtypeStruct((N_CORES, n_acc, d), jnp.float32),
        mesh=_sc_mesh(),
        scratch_types=[
            pltpu.VMEM((CHUNK,), jnp.int32),
            pltpu.VMEM((CHUNK,), jnp.int32),
            pltpu.VMEM((CHUNK, d), jnp.float32),
            pltpu.VMEM_SHARED((n_acc, d), jnp.float32),
            pltpu.SemaphoreType.DMA,
        ],
    )
    def scat_kernel(row_hbm, col_hbm, z_hbm, out_hbm,
                    ridx_v, cidx_v, rows_v, acc_sh, sem):
        c = lax.axis_index("c")
        s = lax.axis_index("s")

        def fill(k, _):
            i = k // (d // LANES)
            j = k % (d // LANES)
            rows_v[i, pl.ds(j * LANES, LANES)] = jnp.zeros((LANES,), jnp.float32)
            return 0

        lax.fori_loop(0, CHUNK * (d // LANES), fill, 0)
        for t in range(zero_blocks):
            pltpu.sync_copy(
                rows_v, acc_sh.at[pl.ds(s * rows_per_sub + t * CHUNK, CHUNK)]
            )
        plsc.subcore_barrier()

        w = c * N_SUBCORES + s

        def body(j, _):
            base = w * (chunks_per_worker * CHUNK) + j * CHUNK
            pltpu.sync_copy(row_hbm.at[pl.ds(base, CHUNK)], ridx_v)
            pltpu.sync_copy(col_hbm.at[pl.ds(base, CHUNK)], cidx_v)
            pltpu.async_copy(z_hbm.at[ridx_v], rows_v, sem).wait()
            pltpu.sync_copy(rows_v, acc_sh.at[cidx_v], add=True)
            return 0

        lax.fori_loop(0, chunks_per_worker, body, 0)
        plsc.subcore_barrier()
        pltpu.sync_copy(
            acc_sh.at[pl.ds(s * rows_per_sub, rows_per_sub)],
            out_hbm.at[c, pl.ds(s * rows_per_sub, rows_per_sub)],
        )

    return scat_kernel


def _linear_body(x_ref, w_ref, degt_ref, z_ref):
    deg = degt_ref[:, 0:1] + degt_ref[:, 1:2] + 1.0
    dinv = lax.rsqrt(deg)
    y = x_ref[...] * dinv
    z_ref[...] = lax.dot_general(
        y, w_ref[...], (((1,), (1,)), ((), ())),
        preferred_element_type=jnp.float32,
    )


def _final_body(s0_ref, s1_ref, z_ref, degt_ref, b_ref, o_ref):
    deg = degt_ref[:, 0:1] + degt_ref[:, 1:2] + 1.0
    dinv = lax.rsqrt(deg)
    o_ref[...] = dinv * (s0_ref[...] + s1_ref[...] + z_ref[...]) + b_ref[...]


def kernel(x, edge_index, W, b):
    n, d = x.shape
    e = edge_index.shape[1]

    row = edge_index[0].astype(jnp.int32)
    col = edge_index[1].astype(jnp.int32)

    per_worker = N_WORKERS * CHUNK
    chunks_per_worker = -(-e // per_worker)
    e_pad = chunks_per_worker * per_worker
    pad = e_pad - e
    row_p = jnp.concatenate([row, jnp.zeros((pad,), jnp.int32)])
    col_p = jnp.concatenate([col, jnp.full((pad,), n, jnp.int32)])

    slab = N_SUBCORES * CHUNK
    n_acc = slab * (-(-(n + 1) // slab))
    rows_per_sub = n_acc // N_SUBCORES

    deg_kernel = _make_deg_kernel(e_pad, chunks_per_worker, n_acc, rows_per_sub)
    degp = deg_kernel(col_p)
    degt = degp[:, :n].T

    z = pl.pallas_call(
        _linear_body,
        out_shape=jax.ShapeDtypeStruct((n, d), jnp.float32),
    )(x, W, degt)

    scat_kernel = _make_scatter_kernel(
        e_pad, chunks_per_worker, n_acc, rows_per_sub, d
    )
    s_part = scat_kernel(row_p, col_p, z)

    out = pl.pallas_call(
        _final_body,
        out_shape=jax.ShapeDtypeStruct((n, d), jnp.float32),
    )(s_part[0, :n, :], s_part[1, :n, :], z, degt, b.reshape(1, d))
    return out


# speedup vs baseline: 41.8351x; 2.5877x over previous
import functools

import jax
import jax.numpy as jnp
from jax import lax
from jax.experimental import pallas as pl
from jax.experimental.pallas import tpu as pltpu
from jax.experimental.pallas import tpu_sc as plsc

N_CORES = 2
N_SUBCORES = 16
N_WORKERS = N_CORES * N_SUBCORES
CHUNK = 64
LANES = 16
NBUF = 5
GDEPTH = 2
NIDX = 10


def _sc_mesh():
    return plsc.VectorSubcoreMesh(core_axis_name="c", subcore_axis_name="s")


def _make_deg_kernel(chunks_per_worker, n_acc):
    C = chunks_per_worker

    @functools.partial(
        pl.kernel,
        out_type=jax.ShapeDtypeStruct((N_WORKERS, n_acc), jnp.float32),
        mesh=_sc_mesh(),
        compiler_params=pltpu.CompilerParams(needs_layout_passes=False),
        scratch_types=[
            pltpu.VMEM((C * CHUNK,), jnp.int32),
            pltpu.VMEM((n_acc,), jnp.float32),
        ],
    )
    def deg_kernel(col_hbm, out_hbm, cidx_v, hist_v):
        c = lax.axis_index("c")
        s = lax.axis_index("s")
        w = c * N_SUBCORES + s

        def zero(i, _):
            hist_v[pl.ds(i * LANES, LANES)] = jnp.zeros((LANES,), jnp.float32)
            return 0

        lax.fori_loop(0, n_acc // LANES, zero, 0)
        pltpu.sync_copy(col_hbm.at[pl.ds(w * C * CHUNK, C * CHUNK)], cidx_v)

        ones = jnp.ones((LANES,), jnp.float32)

        def body(k, _):
            idx = cidx_v[pl.ds(k * LANES, LANES)]
            plsc.addupdate_scatter(hist_v, [idx], ones)
            return 0

        lax.fori_loop(0, C * CHUNK // LANES, body, 0)
        pltpu.sync_copy(hist_v, out_hbm.at[w])

    return deg_kernel


def _make_scatter_kernel(chunks_per_worker, n_acc, rows_per_sub, d):
    C = chunks_per_worker
    assert C % NIDX == 0 and C // NIDX >= 3

    @functools.partial(
        pl.kernel,
        out_type=jax.ShapeDtypeStruct((N_CORES, n_acc, d), jnp.float32),
        mesh=_sc_mesh(),
        scratch_types=[
            [pltpu.VMEM((CHUNK,), jnp.int32) for _ in range(NIDX)],
            [pltpu.VMEM((CHUNK,), jnp.int32) for _ in range(NIDX)],
            [pltpu.VMEM((CHUNK, d), jnp.float32) for _ in range(NBUF)],
            pltpu.VMEM_SHARED((n_acc, d), jnp.float32),
            [pltpu.SemaphoreType.DMA for _ in range(NBUF)],
            [pltpu.SemaphoreType.DMA for _ in range(NBUF)],
            [pltpu.SemaphoreType.DMA for _ in range(NIDX)],
            [pltpu.SemaphoreType.DMA for _ in range(NIDX)],
        ],
    )
    def scat_kernel(row_hbm, col_hbm, z_hbm, out_hbm,
                    rib, cib, rows, acc_sh, gsem, ssem, risem, cisem):
        c = lax.axis_index("c")
        s = lax.axis_index("s")
        w = c * N_SUBCORES + s

        def fill(k, _):
            i = k // (d // LANES)
            jj = k % (d // LANES)
            rows[0][i, pl.ds(jj * LANES, LANES)] = jnp.zeros(
                (LANES,), jnp.float32
            )
            return 0

        lax.fori_loop(0, CHUNK * (d // LANES), fill, 0)
        full = rows_per_sub // CHUNK
        rem = rows_per_sub % CHUNK
        base_row = s * rows_per_sub
        for t in range(full):
            pltpu.sync_copy(
                rows[0], acc_sh.at[pl.ds(base_row + t * CHUNK, CHUNK)]
            )
        if rem:
            pltpu.sync_copy(
                rows[0].at[pl.ds(0, rem)],
                acc_sh.at[pl.ds(base_row + full * CHUNK, rem)],
            )
        plsc.subcore_barrier()

        def start_gather(ui, u):
            pltpu.async_copy(z_hbm.at[rib[ui]], rows[u], gsem[u])

        def wait_gather(u):
            pltpu.make_async_copy(
                z_hbm.at[rib[0]], rows[u], gsem[u]
            ).wait()

        def start_ridx(j, ui):
            pltpu.async_copy(
                row_hbm.at[pl.ds(pl.multiple_of(j * CHUNK, 8), CHUNK)],
                rib[ui], risem[ui],
            )

        def wait_ridx(ui):
            pltpu.make_async_copy(
                row_hbm.at[pl.ds(0, CHUNK)], rib[ui], risem[ui]
            ).wait()

        def start_cidx(j, ui):
            pltpu.async_copy(
                col_hbm.at[pl.ds(pl.multiple_of(j * CHUNK, 8), CHUNK)],
                cib[ui], cisem[ui],
            )

        def wait_cidx(ui):
            pltpu.make_async_copy(
                col_hbm.at[pl.ds(0, CHUNK)], cib[ui], cisem[ui]
            ).wait()

        def start_scatter(u, ui):
            pltpu.async_copy(rows[u], acc_sh.at[cib[ui]], ssem[u], add=True)

        def wait_scatter(u):
            pltpu.make_async_copy(
                rows[u], acc_sh.at[cib[0]], ssem[u]
            ).wait()

        cbase = w * C
        sdepth = NBUF - GDEPTH
        K = NIDX - sdepth

        def emit(j, u, ui, no_swait=False, no_idx=False, no_gather=False):
            wait_gather(u)
            wait_cidx(ui)
            start_scatter(u, ui)
            if not no_swait:
                wait_scatter((u + GDEPTH) % NBUF)
            if not no_idx:
                start_ridx(cbase + j + K, (ui + K) % NIDX)
                start_cidx(cbase + j + K, (ui + K) % NIDX)
            if not no_gather:
                wait_ridx((ui + GDEPTH) % NIDX)
                start_gather((ui + GDEPTH) % NIDX, (u + GDEPTH) % NBUF)

        for t in range(K):
            start_ridx(cbase + t, t)
            start_cidx(cbase + t, t)
        for t in range(GDEPTH):
            wait_ridx(t)
            start_gather(t, t)

        for j in range(NIDX):
            emit(j, j % NBUF, j, no_swait=(j < sdepth))

        n_groups = C // NIDX

        def group(g, _):
            for uu in range(NIDX):
                j = g * NIDX + uu
                emit(j, uu % NBUF, uu)
            return 0

        lax.fori_loop(1, n_groups - 1, group, 0)

        for uu in range(NIDX):
            j = C - NIDX + uu
            emit(j, uu % NBUF, uu,
                 no_idx=(j + K >= C), no_gather=(j + GDEPTH >= C))

        for jj in range(C - sdepth, C):
            wait_scatter(jj % NBUF)

        plsc.subcore_barrier()
        pltpu.sync_copy(
            acc_sh.at[pl.ds(base_row, rows_per_sub)],
            out_hbm.at[c, pl.ds(base_row, rows_per_sub)],
        )

    return scat_kernel


def _matmul_body(x_ref, w_ref, z_ref):
    z_ref[...] = lax.dot_general(
        x_ref[...], w_ref[...], (((1,), (1,)), ((), ())),
        preferred_element_type=jnp.float32,
    )


def _scale_body(zp_ref, degt_ref, z_ref):
    deg = jnp.sum(degt_ref[...], axis=1, keepdims=True) + 1.0
    dinv = lax.rsqrt(deg)
    z_ref[...] = zp_ref[...] * dinv


def _make_final_body(n):
    def final_body(s_ref, z_ref, degt_ref, b_ref, o_ref):
        deg = jnp.sum(degt_ref[...], axis=1, keepdims=True) + 1.0
        dinv = lax.rsqrt(deg)
        o_ref[...] = (
            dinv * (s_ref[0, :n, :] + s_ref[1, :n, :] + z_ref[...])
            + b_ref[...]
        )

    return final_body


def kernel(x, edge_index, W, b):
    n, d = x.shape
    e = edge_index.shape[1]

    row = edge_index[0].astype(jnp.int32)
    col = edge_index[1].astype(jnp.int32)

    rows_per_sub = 8 * (-(-(n + 1) // (N_SUBCORES * 8)))
    n_acc = rows_per_sub * N_SUBCORES

    per_round = N_WORKERS * CHUNK * NIDX
    chunks_per_worker = NIDX * (-(-e // per_round))
    e_pad = chunks_per_worker * N_WORKERS * CHUNK
    pad = e_pad - e
    pad_ar = jnp.arange(pad, dtype=jnp.int32)
    row_p = jnp.concatenate([row, pad_ar % n])
    col_p = jnp.concatenate([col, n + pad_ar % (n_acc - n)])

    deg_kernel = _make_deg_kernel(chunks_per_worker, n_acc)
    degp = deg_kernel(col_p)
    degt = degp[:, :n].T

    zp = pl.pallas_call(
        _matmul_body,
        out_shape=jax.ShapeDtypeStruct((n, d), jnp.float32),
    )(x, W)

    z = pl.pallas_call(
        _scale_body,
        out_shape=jax.ShapeDtypeStruct((n, d), jnp.float32),
    )(zp, degt)

    scat_kernel = _make_scatter_kernel(
        chunks_per_worker, n_acc, rows_per_sub, d
    )
    s_part = scat_kernel(row_p, col_p, z)

    out = pl.pallas_call(
        _make_final_body(n),
        out_shape=jax.ShapeDtypeStruct((n, d), jnp.float32),
    )(s_part, z, degt, b.reshape(1, d))
    return out


# speedup vs baseline: 47.2012x; 1.1283x over previous
import functools

import jax
import jax.numpy as jnp
from jax import lax
from jax.experimental import pallas as pl
from jax.experimental.pallas import tpu as pltpu
from jax.experimental.pallas import tpu_sc as plsc

N_CORES = 2
N_SUBCORES = 16
N_WORKERS = N_CORES * N_SUBCORES
CHUNK = 64
LANES = 16
NBUF = 5
GDEPTH = 3
NIDX = 10


def _sc_mesh():
    return plsc.VectorSubcoreMesh(core_axis_name="c", subcore_axis_name="s")


def _make_deg_kernel(chunks_per_worker, n_acc):
    C = chunks_per_worker

    @functools.partial(
        pl.kernel,
        out_type=jax.ShapeDtypeStruct((N_WORKERS, n_acc), jnp.float32),
        mesh=_sc_mesh(),
        compiler_params=pltpu.CompilerParams(needs_layout_passes=False),
        scratch_types=[
            pltpu.VMEM((C * CHUNK,), jnp.int32),
            pltpu.VMEM((n_acc,), jnp.float32),
        ],
    )
    def deg_kernel(col_hbm, out_hbm, cidx_v, hist_v):
        c = lax.axis_index("c")
        s = lax.axis_index("s")
        w = c * N_SUBCORES + s

        def zero(i, _):
            hist_v[pl.ds(i * LANES, LANES)] = jnp.zeros((LANES,), jnp.float32)
            return 0

        lax.fori_loop(0, n_acc // LANES, zero, 0)
        pltpu.sync_copy(col_hbm.at[pl.ds(w * C * CHUNK, C * CHUNK)], cidx_v)

        ones = jnp.ones((LANES,), jnp.float32)

        def body(k, _):
            idx = cidx_v[pl.ds(k * LANES, LANES)]
            plsc.addupdate_scatter(hist_v, [idx], ones)
            return 0

        lax.fori_loop(0, C * CHUNK // LANES, body, 0)
        pltpu.sync_copy(hist_v, out_hbm.at[w])

    return deg_kernel


def _make_scatter_kernel(chunks_per_worker, n_acc, rows_per_sub, d):
    C = chunks_per_worker
    assert C % NIDX == 0 and C // NIDX >= 3

    @functools.partial(
        pl.kernel,
        out_type=jax.ShapeDtypeStruct((N_CORES, n_acc, d), jnp.float32),
        mesh=_sc_mesh(),
        scratch_types=[
            [pltpu.VMEM((CHUNK,), jnp.int32) for _ in range(NIDX)],
            [pltpu.VMEM((CHUNK,), jnp.int32) for _ in range(NIDX)],
            [pltpu.VMEM((CHUNK, d), jnp.float32) for _ in range(NBUF)],
            pltpu.VMEM_SHARED((n_acc, d), jnp.float32),
            [pltpu.SemaphoreType.DMA for _ in range(NBUF)],
            [pltpu.SemaphoreType.DMA for _ in range(NBUF)],
            [pltpu.SemaphoreType.DMA for _ in range(NIDX)],
            [pltpu.SemaphoreType.DMA for _ in range(NIDX)],
        ],
    )
    def scat_kernel(row_hbm, col_hbm, z_hbm, out_hbm,
                    rib, cib, rows, acc_sh, gsem, ssem, risem, cisem):
        c = lax.axis_index("c")
        s = lax.axis_index("s")
        w = c * N_SUBCORES + s

        def fill(k, _):
            i = k // (d // LANES)
            jj = k % (d // LANES)
            rows[0][i, pl.ds(jj * LANES, LANES)] = jnp.zeros(
                (LANES,), jnp.float32
            )
            return 0

        lax.fori_loop(0, CHUNK * (d // LANES), fill, 0)
        full = rows_per_sub // CHUNK
        rem = rows_per_sub % CHUNK
        base_row = s * rows_per_sub
        for t in range(full):
            pltpu.sync_copy(
                rows[0], acc_sh.at[pl.ds(base_row + t * CHUNK, CHUNK)]
            )
        if rem:
            pltpu.sync_copy(
                rows[0].at[pl.ds(0, rem)],
                acc_sh.at[pl.ds(base_row + full * CHUNK, rem)],
            )
        plsc.subcore_barrier()

        def start_gather(ui, u):
            pltpu.async_copy(z_hbm.at[rib[ui]], rows[u], gsem[u])

        def wait_gather(u):
            pltpu.make_async_copy(
                z_hbm.at[rib[0]], rows[u], gsem[u]
            ).wait()

        def start_ridx(j, ui):
            pltpu.async_copy(
                row_hbm.at[pl.ds(pl.multiple_of(j * CHUNK, 8), CHUNK)],
                rib[ui], risem[ui],
            )

        def wait_ridx(ui):
            pltpu.make_async_copy(
                row_hbm.at[pl.ds(0, CHUNK)], rib[ui], risem[ui]
            ).wait()

        def start_cidx(j, ui):
            pltpu.async_copy(
                col_hbm.at[pl.ds(pl.multiple_of(j * CHUNK, 8), CHUNK)],
                cib[ui], cisem[ui],
            )

        def wait_cidx(ui):
            pltpu.make_async_copy(
                col_hbm.at[pl.ds(0, CHUNK)], cib[ui], cisem[ui]
            ).wait()

        def start_scatter(u, ui):
            pltpu.async_copy(rows[u], acc_sh.at[cib[ui]], ssem[u], add=True)

        def wait_scatter(u):
            pltpu.make_async_copy(
                rows[u], acc_sh.at[cib[0]], ssem[u]
            ).wait()

        cbase = w * C
        sdepth = NBUF - GDEPTH
        K = NIDX - sdepth

        def emit(j, u, ui, no_swait=False, no_idx=False, no_gather=False):
            wait_gather(u)
            wait_cidx(ui)
            start_scatter(u, ui)
            if not no_swait:
                wait_scatter((u + GDEPTH) % NBUF)
            if not no_idx:
                start_ridx(cbase + j + K, (ui + K) % NIDX)
                start_cidx(cbase + j + K, (ui + K) % NIDX)
            if not no_gather:
                wait_ridx((ui + GDEPTH) % NIDX)
                start_gather((ui + GDEPTH) % NIDX, (u + GDEPTH) % NBUF)

        for t in range(K):
            start_ridx(cbase + t, t)
            start_cidx(cbase + t, t)
        for t in range(GDEPTH):
            wait_ridx(t)
            start_gather(t, t)

        for j in range(NIDX):
            emit(j, j % NBUF, j, no_swait=(j < sdepth))

        n_groups = C // NIDX

        def group(g, _):
            for uu in range(NIDX):
                j = g * NIDX + uu
                emit(j, uu % NBUF, uu)
            return 0

        lax.fori_loop(1, n_groups - 1, group, 0)

        for uu in range(NIDX):
            j = C - NIDX + uu
            emit(j, uu % NBUF, uu,
                 no_idx=(j + K >= C), no_gather=(j + GDEPTH >= C))

        for jj in range(C - sdepth, C):
            wait_scatter(jj % NBUF)

        plsc.subcore_barrier()
        pltpu.sync_copy(
            acc_sh.at[pl.ds(base_row, rows_per_sub)],
            out_hbm.at[c, pl.ds(base_row, rows_per_sub)],
        )

    return scat_kernel


def _linear_body(x_ref, w_ref, degt_ref, z_ref):
    deg = jnp.sum(degt_ref[...], axis=1, keepdims=True) + 1.0
    dinv = lax.rsqrt(deg)
    y = x_ref[...] * dinv
    z_ref[...] = lax.dot_general(
        y, w_ref[...], (((1,), (1,)), ((), ())),
        preferred_element_type=jnp.float32,
    )


def _make_final_body(n):
    def final_body(s_ref, z_ref, degt_ref, b_ref, o_ref):
        deg = jnp.sum(degt_ref[...], axis=1, keepdims=True) + 1.0
        dinv = lax.rsqrt(deg)
        o_ref[...] = (
            dinv * (s_ref[0, :n, :] + s_ref[1, :n, :] + z_ref[...])
            + b_ref[...]
        )

    return final_body


def kernel(x, edge_index, W, b):
    n, d = x.shape
    e = edge_index.shape[1]

    row = edge_index[0].astype(jnp.int32)
    col = edge_index[1].astype(jnp.int32)

    rows_per_sub = 8 * (-(-(n + 1) // (N_SUBCORES * 8)))
    n_acc = rows_per_sub * N_SUBCORES

    per_round = N_WORKERS * CHUNK * NIDX
    chunks_per_worker = NIDX * (-(-e // per_round))
    e_pad = chunks_per_worker * N_WORKERS * CHUNK
    pad = e_pad - e
    pad_ar = jnp.arange(pad, dtype=jnp.int32)
    row_p = jnp.concatenate([row, pad_ar % n])
    col_p = jnp.concatenate([col, n + pad_ar % (n_acc - n)])

    deg_kernel = _make_deg_kernel(chunks_per_worker, n_acc)
    degp = deg_kernel(col_p)
    degt = degp[:, :n].T

    z = pl.pallas_call(
        _linear_body,
        out_shape=jax.ShapeDtypeStruct((n, d), jnp.float32),
    )(x, W, degt)

    scat_kernel = _make_scatter_kernel(
        chunks_per_worker, n_acc, rows_per_sub, d
    )
    s_part = scat_kernel(row_p, col_p, z)

    out = pl.pallas_call(
        _make_final_body(n),
        out_shape=jax.ShapeDtypeStruct((n, d), jnp.float32),
    )(s_part, z, degt, b.reshape(1, d))
    return out


# speedup vs baseline: 47.7285x; 1.0112x over previous
import functools

import jax
import jax.numpy as jnp
from jax import lax
from jax.experimental import pallas as pl
from jax.experimental.pallas import tpu as pltpu
from jax.experimental.pallas import tpu_sc as plsc

N_CORES = 2
N_SUBCORES = 16
N_WORKERS = N_CORES * N_SUBCORES
CHUNK = 72
LANES = 16
NBUF = 5
GDEPTH = 3
NIDX = 10


def _sc_mesh():
    return plsc.VectorSubcoreMesh(core_axis_name="c", subcore_axis_name="s")


def _make_deg_kernel(chunks_per_worker, n_acc):
    C = chunks_per_worker

    @functools.partial(
        pl.kernel,
        out_type=jax.ShapeDtypeStruct((N_WORKERS, n_acc), jnp.float32),
        mesh=_sc_mesh(),
        compiler_params=pltpu.CompilerParams(needs_layout_passes=False),
        scratch_types=[
            pltpu.VMEM((C * CHUNK,), jnp.int32),
            pltpu.VMEM((n_acc,), jnp.float32),
        ],
    )
    def deg_kernel(col_hbm, out_hbm, cidx_v, hist_v):
        c = lax.axis_index("c")
        s = lax.axis_index("s")
        w = c * N_SUBCORES + s

        def zero(i, _):
            hist_v[pl.ds(i * LANES, LANES)] = jnp.zeros((LANES,), jnp.float32)
            return 0

        lax.fori_loop(0, n_acc // LANES, zero, 0)
        pltpu.sync_copy(col_hbm.at[pl.ds(w * C * CHUNK, C * CHUNK)], cidx_v)

        ones = jnp.ones((LANES,), jnp.float32)

        def body(k, _):
            idx = cidx_v[pl.ds(k * LANES, LANES)]
            plsc.addupdate_scatter(hist_v, [idx], ones)
            return 0

        lax.fori_loop(0, C * CHUNK // LANES, body, 0)
        pltpu.sync_copy(hist_v, out_hbm.at[w])

    return deg_kernel


def _make_scatter_kernel(chunks_per_worker, n_acc, rows_per_sub, d):
    C = chunks_per_worker
    assert C % NIDX == 0 and C // NIDX >= 3

    @functools.partial(
        pl.kernel,
        out_type=jax.ShapeDtypeStruct((N_CORES, n_acc, d), jnp.float32),
        mesh=_sc_mesh(),
        scratch_types=[
            [pltpu.VMEM((CHUNK,), jnp.int32) for _ in range(NIDX)],
            [pltpu.VMEM((CHUNK,), jnp.int32) for _ in range(NIDX)],
            [pltpu.VMEM((CHUNK, d), jnp.float32) for _ in range(NBUF)],
            pltpu.VMEM_SHARED((n_acc, d), jnp.float32),
            [pltpu.SemaphoreType.DMA for _ in range(NBUF)],
            [pltpu.SemaphoreType.DMA for _ in range(NBUF)],
            [pltpu.SemaphoreType.DMA for _ in range(NIDX)],
            [pltpu.SemaphoreType.DMA for _ in range(NIDX)],
        ],
    )
    def scat_kernel(row_hbm, col_hbm, z_hbm, out_hbm,
                    rib, cib, rows, acc_sh, gsem, ssem, risem, cisem):
        c = lax.axis_index("c")
        s = lax.axis_index("s")
        w = c * N_SUBCORES + s

        def fill(k, _):
            i = k // (d // LANES)
            jj = k % (d // LANES)
            rows[0][i, pl.ds(jj * LANES, LANES)] = jnp.zeros(
                (LANES,), jnp.float32
            )
            return 0

        lax.fori_loop(0, CHUNK * (d // LANES), fill, 0)
        full = rows_per_sub // CHUNK
        rem = rows_per_sub % CHUNK
        base_row = s * rows_per_sub
        for t in range(full):
            pltpu.sync_copy(
                rows[0], acc_sh.at[pl.ds(base_row + t * CHUNK, CHUNK)]
            )
        if rem:
            pltpu.sync_copy(
                rows[0].at[pl.ds(0, rem)],
                acc_sh.at[pl.ds(base_row + full * CHUNK, rem)],
            )
        plsc.subcore_barrier()

        def start_gather(ui, u):
            pltpu.async_copy(z_hbm.at[rib[ui]], rows[u], gsem[u])

        def wait_gather(u):
            pltpu.make_async_copy(
                z_hbm.at[rib[0]], rows[u], gsem[u]
            ).wait()

        def start_ridx(j, ui):
            pltpu.async_copy(
                row_hbm.at[pl.ds(pl.multiple_of(j * CHUNK, 8), CHUNK)],
                rib[ui], risem[ui],
            )

        def wait_ridx(ui):
            pltpu.make_async_copy(
                row_hbm.at[pl.ds(0, CHUNK)], rib[ui], risem[ui]
            ).wait()

        def start_cidx(j, ui):
            pltpu.async_copy(
                col_hbm.at[pl.ds(pl.multiple_of(j * CHUNK, 8), CHUNK)],
                cib[ui], cisem[ui],
            )

        def wait_cidx(ui):
            pltpu.make_async_copy(
                col_hbm.at[pl.ds(0, CHUNK)], cib[ui], cisem[ui]
            ).wait()

        def start_scatter(u, ui):
            pltpu.async_copy(rows[u], acc_sh.at[cib[ui]], ssem[u], add=True)

        def wait_scatter(u):
            pltpu.make_async_copy(
                rows[u], acc_sh.at[cib[0]], ssem[u]
            ).wait()

        cbase = w * C
        sdepth = NBUF - GDEPTH
        K = NIDX - sdepth

        def emit(j, u, ui, no_swait=False, no_idx=False, no_gather=False):
            wait_gather(u)
            wait_cidx(ui)
            start_scatter(u, ui)
            if not no_swait:
                wait_scatter((u + GDEPTH) % NBUF)
            if not no_idx:
                start_ridx(cbase + j + K, (ui + K) % NIDX)
                start_cidx(cbase + j + K, (ui + K) % NIDX)
            if not no_gather:
                wait_ridx((ui + GDEPTH) % NIDX)
                start_gather((ui + GDEPTH) % NIDX, (u + GDEPTH) % NBUF)

        for t in range(K):
            start_ridx(cbase + t, t)
            start_cidx(cbase + t, t)
        for t in range(GDEPTH):
            wait_ridx(t)
            start_gather(t, t)

        for j in range(NIDX):
            emit(j, j % NBUF, j, no_swait=(j < sdepth))

        n_groups = C // NIDX

        def group(g, _):
            for uu in range(NIDX):
                j = g * NIDX + uu
                emit(j, uu % NBUF, uu)
            return 0

        lax.fori_loop(1, n_groups - 1, group, 0)

        for uu in range(NIDX):
            j = C - NIDX + uu
            emit(j, uu % NBUF, uu,
                 no_idx=(j + K >= C), no_gather=(j + GDEPTH >= C))

        for jj in range(C - sdepth, C):
            wait_scatter(jj % NBUF)

        plsc.subcore_barrier()
        pltpu.sync_copy(
            acc_sh.at[pl.ds(base_row, rows_per_sub)],
            out_hbm.at[c, pl.ds(base_row, rows_per_sub)],
        )

    return scat_kernel


def _linear_body(x_ref, w_ref, degt_ref, z_ref):
    deg = jnp.sum(degt_ref[...], axis=1, keepdims=True) + 1.0
    dinv = lax.rsqrt(deg)
    y = x_ref[...] * dinv
    z_ref[...] = lax.dot_general(
        y, w_ref[...], (((1,), (1,)), ((), ())),
        preferred_element_type=jnp.float32,
    )


def _make_final_body(n):
    def final_body(s_ref, z_ref, degt_ref, b_ref, o_ref):
        deg = jnp.sum(degt_ref[...], axis=1, keepdims=True) + 1.0
        dinv = lax.rsqrt(deg)
        o_ref[...] = (
            dinv * (s_ref[0, :n, :] + s_ref[1, :n, :] + z_ref[...])
            + b_ref[...]
        )

    return final_body


def kernel(x, edge_index, W, b):
    n, d = x.shape
    e = edge_index.shape[1]

    row = edge_index[0].astype(jnp.int32)
    col = edge_index[1].astype(jnp.int32)

    rows_per_sub = 8 * (-(-(n + 1) // (N_SUBCORES * 8)))
    n_acc = rows_per_sub * N_SUBCORES

    per_round = N_WORKERS * CHUNK * NIDX
    chunks_per_worker = NIDX * (-(-e // per_round))
    e_pad = chunks_per_worker * N_WORKERS * CHUNK
    pad = e_pad - e
    pad_ar = jnp.arange(pad, dtype=jnp.int32)
    row_p = jnp.concatenate([row, pad_ar % n])
    col_p = jnp.concatenate([col, n + pad_ar % (n_acc - n)])

    deg_kernel = _make_deg_kernel(chunks_per_worker, n_acc)
    degp = deg_kernel(col_p)
    degt = degp[:, :n].T

    z = pl.pallas_call(
        _linear_body,
        out_shape=jax.ShapeDtypeStruct((n, d), jnp.float32),
    )(x, W, degt)

    scat_kernel = _make_scatter_kernel(
        chunks_per_worker, n_acc, rows_per_sub, d
    )
    s_part = scat_kernel(row_p, col_p, z)

    out = pl.pallas_call(
        _make_final_body(n),
        out_shape=jax.ShapeDtypeStruct((n, d), jnp.float32),
    )(s_part, z, degt, b.reshape(1, d))
    return out


# speedup vs baseline: 48.5704x; 1.0176x over previous
import functools

import jax
import jax.numpy as jnp
from jax import lax
from jax.experimental import pallas as pl
from jax.experimental.pallas import tpu as pltpu
from jax.experimental.pallas import tpu_sc as plsc

N_CORES = 2
N_SUBCORES = 16
N_WORKERS = N_CORES * N_SUBCORES
CHUNK = 72
LANES = 16
NBUF = 5
GDEPTH = 4
NIDX = 10


def _sc_mesh():
    return plsc.VectorSubcoreMesh(core_axis_name="c", subcore_axis_name="s")


def _make_deg_kernel(chunks_per_worker, n_acc):
    C = chunks_per_worker

    @functools.partial(
        pl.kernel,
        out_type=jax.ShapeDtypeStruct((N_WORKERS, n_acc), jnp.float32),
        mesh=_sc_mesh(),
        compiler_params=pltpu.CompilerParams(needs_layout_passes=False),
        scratch_types=[
            pltpu.VMEM((C * CHUNK,), jnp.int32),
            pltpu.VMEM((n_acc,), jnp.float32),
        ],
    )
    def deg_kernel(col_hbm, out_hbm, cidx_v, hist_v):
        c = lax.axis_index("c")
        s = lax.axis_index("s")
        w = c * N_SUBCORES + s

        def zero(i, _):
            hist_v[pl.ds(i * LANES, LANES)] = jnp.zeros((LANES,), jnp.float32)
            return 0

        lax.fori_loop(0, n_acc // LANES, zero, 0)
        pltpu.sync_copy(col_hbm.at[pl.ds(w * C * CHUNK, C * CHUNK)], cidx_v)

        ones = jnp.ones((LANES,), jnp.float32)

        def body(k, _):
            idx = cidx_v[pl.ds(k * LANES, LANES)]
            plsc.addupdate_scatter(hist_v, [idx], ones)
            return 0

        lax.fori_loop(0, C * CHUNK // LANES, body, 0)
        pltpu.sync_copy(hist_v, out_hbm.at[w])

    return deg_kernel


def _make_scatter_kernel(chunks_per_worker, n_acc, rows_per_sub, d):
    C = chunks_per_worker
    assert C % NIDX == 0 and C // NIDX >= 3

    @functools.partial(
        pl.kernel,
        out_type=jax.ShapeDtypeStruct((N_CORES, n_acc, d), jnp.float32),
        mesh=_sc_mesh(),
        scratch_types=[
            [pltpu.VMEM((CHUNK,), jnp.int32) for _ in range(NIDX)],
            [pltpu.VMEM((CHUNK,), jnp.int32) for _ in range(NIDX)],
            [pltpu.VMEM((CHUNK, d), jnp.float32) for _ in range(NBUF)],
            pltpu.VMEM_SHARED((n_acc, d), jnp.float32),
            [pltpu.SemaphoreType.DMA for _ in range(NBUF)],
            [pltpu.SemaphoreType.DMA for _ in range(NBUF)],
            [pltpu.SemaphoreType.DMA for _ in range(NIDX)],
            [pltpu.SemaphoreType.DMA for _ in range(NIDX)],
        ],
    )
    def scat_kernel(row_hbm, col_hbm, z_hbm, out_hbm,
                    rib, cib, rows, acc_sh, gsem, ssem, risem, cisem):
        c = lax.axis_index("c")
        s = lax.axis_index("s")
        w = c * N_SUBCORES + s

        def fill(k, _):
            i = k // (d // LANES)
            jj = k % (d // LANES)
            rows[0][i, pl.ds(jj * LANES, LANES)] = jnp.zeros(
                (LANES,), jnp.float32
            )
            return 0

        lax.fori_loop(0, CHUNK * (d // LANES), fill, 0)
        full = rows_per_sub // CHUNK
        rem = rows_per_sub % CHUNK
        base_row = s * rows_per_sub
        for t in range(full):
            pltpu.sync_copy(
                rows[0], acc_sh.at[pl.ds(base_row + t * CHUNK, CHUNK)]
            )
        if rem:
            pltpu.sync_copy(
                rows[0].at[pl.ds(0, rem)],
                acc_sh.at[pl.ds(base_row + full * CHUNK, rem)],
            )
        plsc.subcore_barrier()

        def start_gather(ui, u):
            pltpu.async_copy(z_hbm.at[rib[ui]], rows[u], gsem[u])

        def wait_gather(u):
            pltpu.make_async_copy(
                z_hbm.at[rib[0]], rows[u], gsem[u]
            ).wait()

        def start_ridx(j, ui):
            pltpu.async_copy(
                row_hbm.at[pl.ds(pl.multiple_of(j * CHUNK, 8), CHUNK)],
                rib[ui], risem[ui],
            )

        def wait_ridx(ui):
            pltpu.make_async_copy(
                row_hbm.at[pl.ds(0, CHUNK)], rib[ui], risem[ui]
            ).wait()

        def start_cidx(j, ui):
            pltpu.async_copy(
                col_hbm.at[pl.ds(pl.multiple_of(j * CHUNK, 8), CHUNK)],
                cib[ui], cisem[ui],
            )

        def wait_cidx(ui):
            pltpu.make_async_copy(
                col_hbm.at[pl.ds(0, CHUNK)], cib[ui], cisem[ui]
            ).wait()

        def start_scatter(u, ui):
            pltpu.async_copy(rows[u], acc_sh.at[cib[ui]], ssem[u], add=True)

        def wait_scatter(u):
            pltpu.make_async_copy(
                rows[u], acc_sh.at[cib[0]], ssem[u]
            ).wait()

        cbase = w * C
        sdepth = NBUF - GDEPTH
        K = NIDX - sdepth

        def emit(j, u, ui, no_swait=False, no_idx=False, no_gather=False):
            wait_gather(u)
            wait_cidx(ui)
            start_scatter(u, ui)
            if not no_swait:
                wait_scatter((u + GDEPTH) % NBUF)
            if not no_idx:
                start_ridx(cbase + j + K, (ui + K) % NIDX)
                start_cidx(cbase + j + K, (ui + K) % NIDX)
            if not no_gather:
                wait_ridx((ui + GDEPTH) % NIDX)
                start_gather((ui + GDEPTH) % NIDX, (u + GDEPTH) % NBUF)

        for t in range(K):
            start_ridx(cbase + t, t)
            start_cidx(cbase + t, t)
        for t in range(GDEPTH):
            wait_ridx(t)
            start_gather(t, t)

        for j in range(NIDX):
            emit(j, j % NBUF, j, no_swait=(j < sdepth))

        n_groups = C // NIDX

        def group(g, _):
            for uu in range(NIDX):
                j = g * NIDX + uu
                emit(j, uu % NBUF, uu)
            return 0

        lax.fori_loop(1, n_groups - 1, group, 0)

        for uu in range(NIDX):
            j = C - NIDX + uu
            emit(j, uu % NBUF, uu,
                 no_idx=(j + K >= C), no_gather=(j + GDEPTH >= C))

        for jj in range(C - sdepth, C):
            wait_scatter(jj % NBUF)

        plsc.subcore_barrier()
        pltpu.sync_copy(
            acc_sh.at[pl.ds(base_row, rows_per_sub)],
            out_hbm.at[c, pl.ds(base_row, rows_per_sub)],
        )

    return scat_kernel


def _linear_body(x_ref, w_ref, degt_ref, z_ref):
    deg = jnp.sum(degt_ref[...], axis=1, keepdims=True) + 1.0
    dinv = lax.rsqrt(deg)
    y = x_ref[...] * dinv
    z_ref[...] = lax.dot_general(
        y, w_ref[...], (((1,), (1,)), ((), ())),
        preferred_element_type=jnp.float32,
    )


def _make_final_body(n):
    def final_body(s_ref, z_ref, degt_ref, b_ref, o_ref):
        deg = jnp.sum(degt_ref[...], axis=1, keepdims=True) + 1.0
        dinv = lax.rsqrt(deg)
        o_ref[...] = (
            dinv * (s_ref[0, :n, :] + s_ref[1, :n, :] + z_ref[...])
            + b_ref[...]
        )

    return final_body


def kernel(x, edge_index, W, b):
    n, d = x.shape
    e = edge_index.shape[1]

    row = edge_index[0].astype(jnp.int32)
    col = edge_index[1].astype(jnp.int32)

    rows_per_sub = 8 * (-(-(n + 1) // (N_SUBCORES * 8)))
    n_acc = rows_per_sub * N_SUBCORES

    per_round = N_WORKERS * CHUNK * NIDX
    chunks_per_worker = NIDX * (-(-e // per_round))
    e_pad = chunks_per_worker * N_WORKERS * CHUNK
    pad = e_pad - e
    pad_ar = jnp.arange(pad, dtype=jnp.int32)
    row_p = jnp.concatenate([row, pad_ar % n])
    col_p = jnp.concatenate([col, n + pad_ar % (n_acc - n)])

    deg_kernel = _make_deg_kernel(chunks_per_worker, n_acc)
    degp = deg_kernel(col_p)
    degt = degp[:, :n].T

    z = pl.pallas_call(
        _linear_body,
        out_shape=jax.ShapeDtypeStruct((n, d), jnp.float32),
    )(x, W, degt)

    scat_kernel = _make_scatter_kernel(
        chunks_per_worker, n_acc, rows_per_sub, d
    )
    s_part = scat_kernel(row_p, col_p, z)

    out = pl.pallas_call(
        _make_final_body(n),
        out_shape=jax.ShapeDtypeStruct((n, d), jnp.float32),
    )(s_part, z, degt, b.reshape(1, d))
    return out
